# baseline (speedup 1.0000x reference)
import functools

import jax
import jax.numpy as jnp
from jax import lax
from jax.experimental import pallas as pl
from jax.experimental.pallas import tpu as pltpu
from jax.experimental.pallas import tpu_sc as plsc

_EPS = 0.01
_K = 4097
_NCH = 32


def _toz_body(x_hbm, out_hbm, cb, sem):
    wid = lax.axis_index("s") * 2 + lax.axis_index("c")
    a0 = 2 * wid

    zv = jnp.zeros((16,), jnp.float32)
    lane = lax.iota(jnp.int32, 16)

    def _zero_grp(j, carry):
        b = j // 8
        g = lax.rem(j, 8) * 16
        for s in range(4):
            cb[s, 0, b, pl.ds(g, 16)] = zv
        return carry

    lax.fori_loop(0, 512, _zero_grp, 0)

    for a in range(2):
        for b in range(64):
            l = 1 + 64 * a + b
            if l < 128:
                cb[1 + a, 0, b, pl.ds((l // 16) * 16, 16)] = jnp.where(
                    lane == l % 16, _EPS, 0.0)
    cb[3, 0, 63, pl.ds(0, 16)] = jnp.where(lane == 0, _EPS, 0.0)

    copies = []
    for c in range(_NCH):
        for a in range(2):
            dst = out_hbm.at[0, pl.ds(a0 + a, 1), :, pl.ds(128 * c, 128)]

            @pl.when(c == wid)
            def _(dst=dst, a=a):
                pltpu.make_async_copy(cb.at[1 + a], dst, sem).start()

            if a == 1:
                @pl.when(c == wid + 1)
                def _(dst=dst):
                    pltpu.make_async_copy(cb.at[3], dst, sem).start()

                @pl.when(jnp.logical_and(c != wid, c != wid + 1))
                def _(dst=dst):
                    pltpu.make_async_copy(cb.at[0], dst, sem).start()
            else:
                @pl.when(c != wid)
                def _(dst=dst):
                    pltpu.make_async_copy(cb.at[0], dst, sem).start()

            copies.append(pltpu.make_async_copy(cb.at[0], dst, sem))

    for c in copies:
        c.wait()


@functools.partial(jax.jit, static_argnums=())
def kernel(x):
    k = pl.kernel(
        _toz_body,
        out_type=jax.ShapeDtypeStruct((1, 64, 64, _K), jnp.float32),
        mesh=plsc.VectorSubcoreMesh(core_axis_name="c", subcore_axis_name="s"),
        scratch_types=[
            pltpu.VMEM((4, 1, 64, 128), jnp.float32),
            pltpu.SemaphoreType.DMA,
        ],
    )
    out = k(x)
    out = lax.dynamic_update_slice(out, x.reshape(1, 64, 64, 1), (0, 0, 0, 0))
    eps_tail = jnp.full((1, 1, 1, 1), _EPS, jnp.float32)
    out = lax.dynamic_update_slice(out, eps_tail, (0, 63, 63, _K - 1))
    return jnp.transpose(out, (3, 0, 1, 2))
